# baseline (speedup 1.0000x reference)
import functools

import jax
import jax.numpy as jnp
from jax import lax
from jax.experimental import pallas as pl
from jax.experimental.pallas import tpu as pltpu
from jax.experimental.pallas import tpu_sc as plsc

B = 16384
DEG = 32
D = 128
DW = D // 2
N_TAB = 100000

_info = plsc.get_sparse_core_info()
_NC, _NS = _info.num_cores, _info.num_subcores
NW = _NC * _NS
B_PER_W = B // NW
GROUP = 128
N_GROUPS = B_PER_W // GROUP
CHUNK_N = 4
CHUNKS = GROUP // CHUNK_N


def _sc_gather_mean_body(nb_hbm, nodes_hbm, table_hbm, tbl_pk_hbm,
                         out_self, out_neigh,
                         nidx_v, buf0, buf1, acc_v, sidx_v, srows_v,
                         sem0, sem1, sem_s):
    wid = lax.axis_index("s") * _NC + lax.axis_index("c")
    base = wid * B_PER_W

    def fire(j, buf, sem):
        pltpu.async_copy(tbl_pk_hbm.at[nidx_v.at[j]], buf, sem)

    def wait(buf, sem):
        pltpu.make_async_copy(tbl_pk_hbm.at[pl.ds(0, CHUNK_N * DEG)], buf,
                              sem).wait()

    def accumulate(j, buf):
        def node_body(n, _):
            row0 = n * DEG
            cols = [pl.ds(c * 16, 16) for c in range(DW // 16)]
            accs = [plsc.bitcast(buf[row0, sl], jnp.bfloat16) for sl in cols]
            for r in range(1, DEG):
                for c in range(DW // 16):
                    accs[c] = accs[c] + plsc.bitcast(buf[row0 + r, cols[c]],
                                                     jnp.bfloat16)
            for c in range(DW // 16):
                acc_v[j * CHUNK_N + n, cols[c]] = plsc.bitcast(
                    accs[c] * (1.0 / DEG), jnp.int32)
            return 0

        lax.fori_loop(0, CHUNK_N, node_body, 0)

    def group_body(gr, _):
        gbase = base + gr * GROUP

        grow = pl.multiple_of(gbase // CHUNK_N, CHUNKS)
        pltpu.sync_copy(nb_hbm.at[pl.ds(grow, CHUNKS)], nidx_v)

        pltpu.sync_copy(nodes_hbm.at[pl.ds(gbase, GROUP)], sidx_v)
        self_dma = pltpu.async_copy(table_hbm.at[sidx_v], srows_v, sem_s)

        fire(0, buf0, sem0)
        fire(1, buf1, sem1)

        def pair_body(jj, _):
            j = 2 * jj
            wait(buf0, sem0)

            @pl.when(jj < CHUNKS // 2 - 1)
            def _():
                fire(j + 2, buf0, sem0)

            accumulate(j, buf0)
            wait(buf1, sem1)

            @pl.when(jj < CHUNKS // 2 - 1)
            def _():
                fire(j + 3, buf1, sem1)

            accumulate(j + 1, buf1)
            return 0

        lax.fori_loop(0, CHUNKS // 2, pair_body, 0)

        pltpu.sync_copy(acc_v, out_neigh.at[pl.ds(gbase, GROUP)])
        self_dma.wait()
        pltpu.sync_copy(srows_v, out_self.at[pl.ds(gbase, GROUP)])
        return 0

    lax.fori_loop(0, N_GROUPS, group_body, 0)


@functools.partial(
    pl.kernel,
    mesh=plsc.VectorSubcoreMesh(core_axis_name="c", subcore_axis_name="s"),
    compiler_params=pltpu.CompilerParams(needs_layout_passes=False,
                                         use_tc_tiling_on_sc=False),
    out_type=[
        jax.ShapeDtypeStruct((B, D), jnp.float32),
        jax.ShapeDtypeStruct((B, DW), jnp.int32),
    ],
    scratch_types=[
        pltpu.VMEM((CHUNKS, CHUNK_N * DEG), jnp.int32),
        pltpu.VMEM((CHUNK_N * DEG, DW), jnp.int32),
        pltpu.VMEM((CHUNK_N * DEG, DW), jnp.int32),
        pltpu.VMEM((GROUP, DW), jnp.int32),
        pltpu.VMEM((GROUP,), jnp.int32),
        pltpu.VMEM((GROUP, D), jnp.float32),
        pltpu.SemaphoreType.DMA,
        pltpu.SemaphoreType.DMA,
        pltpu.SemaphoreType.DMA,
    ],
)
def _sc_gather_mean(nb_hbm, nodes_hbm, table_hbm, tbl_pk_hbm,
                    out_self, out_neigh,
                    nidx_v, buf0, buf1, acc_v, sidx_v, srows_v,
                    sem0, sem1, sem_s):
    _sc_gather_mean_body(nb_hbm, nodes_hbm, table_hbm, tbl_pk_hbm,
                         out_self, out_neigh,
                         nidx_v, buf0, buf1, acc_v, sidx_v, srows_v,
                         sem0, sem1, sem_s)


CVT_ROWS = 2000


def _tc_convert_body(t_ref, o_ref):
    o_ref[...] = t_ref[...].astype(jnp.bfloat16)


def _tc_dense_body(s_ref, n_ref, w_ref, b_ref, g_ref, be_ref, out_ref):
    s = s_ref[...]
    n = n_ref[...].astype(jnp.float32)
    w = w_ref[...]
    lin = lax.dot_general(s, w[:, :D], (((1,), (1,)), ((), ())),
                          preferred_element_type=jnp.float32)
    lin = lin + lax.dot_general(n, w[:, D:], (((1,), (1,)), ((), ())),
                                preferred_element_type=jnp.float32)
    lin = lin + b_ref[...]
    mu = jnp.mean(lin, axis=0, keepdims=True)
    xc = lin - mu
    var = jnp.mean(xc * xc, axis=0, keepdims=True)
    inv = lax.rsqrt(var + 1e-5)
    out_ref[...] = jnp.maximum(xc * inv * g_ref[...] + be_ref[...], 0.0)


def kernel(nodes, neighbors, emb_table, W1, b1, gamma, beta):
    tbl_bf = pl.pallas_call(
        _tc_convert_body,
        grid=(N_TAB // CVT_ROWS,),
        in_specs=[pl.BlockSpec((CVT_ROWS, D), lambda i: (i, 0))],
        out_specs=pl.BlockSpec((CVT_ROWS, D), lambda i: (i, 0)),
        out_shape=jax.ShapeDtypeStruct((N_TAB, D), jnp.bfloat16),
    )(emb_table)
    tbl_pk = lax.bitcast_convert_type(tbl_bf.reshape(N_TAB, DW, 2), jnp.int32)

    nb_chunked = neighbors.reshape(B // CHUNK_N, CHUNK_N * DEG)
    self_feats, neigh_pk = _sc_gather_mean(nb_chunked, nodes, emb_table,
                                           tbl_pk)
    neigh_mean = lax.bitcast_convert_type(neigh_pk,
                                          jnp.bfloat16).reshape(B, D)
    out = pl.pallas_call(
        _tc_dense_body,
        out_shape=jax.ShapeDtypeStruct((B, D), jnp.float32),
    )(self_feats, neigh_mean, W1,
      b1.reshape(1, D), gamma.reshape(1, D), beta.reshape(1, D))
    return out


# speedup vs baseline: 4.0970x; 4.0970x over previous
import functools

import jax
import jax.numpy as jnp
from jax import lax
from jax.experimental import pallas as pl
from jax.experimental.pallas import tpu as pltpu
from jax.experimental.pallas import tpu_sc as plsc

B = 16384
DEG = 32
D = 128

_info = plsc.get_sparse_core_info()
_NC, _NS = _info.num_cores, _info.num_subcores
NW = _NC * _NS
B_PER_W = B // NW
GROUP = 128
N_GROUPS = B_PER_W // GROUP
CHUNK_N = 4
CHUNKS = GROUP // CHUNK_N


def _sc_gather_mean_body(nb_hbm, nodes_hbm, table_hbm, out_self, out_neigh,
                         nidx_v, buf0, buf1, acc_v, sidx_v, srows_v,
                         sem0, sem1, sem_s):
    wid = lax.axis_index("s") * _NC + lax.axis_index("c")
    base = wid * B_PER_W

    def fire(j, buf, sem):
        pltpu.async_copy(table_hbm.at[nidx_v.at[j]], buf, sem)

    def wait(buf, sem):
        pltpu.make_async_copy(table_hbm.at[pl.ds(0, CHUNK_N * DEG)], buf,
                              sem).wait()

    def accumulate(j, buf):
        def node_body(n, _):
            row0 = n * DEG
            for h in range(2):
                cols = [pl.ds((4 * h + c) * 16, 16) for c in range(4)]
                accs = [buf[row0, sl] for sl in cols]
                for r in range(1, DEG):
                    for c in range(4):
                        accs[c] = accs[c] + buf[row0 + r, cols[c]]
                for c in range(4):
                    acc_v[j * CHUNK_N + n, cols[c]] = accs[c] * (1.0 / DEG)
            return 0

        lax.fori_loop(0, CHUNK_N, node_body, 0)

    def group_body(gr, _):
        gbase = base + gr * GROUP

        grow = pl.multiple_of(gbase // CHUNK_N, CHUNKS)
        pltpu.sync_copy(nb_hbm.at[pl.ds(grow, CHUNKS)], nidx_v)

        pltpu.sync_copy(nodes_hbm.at[pl.ds(gbase, GROUP)], sidx_v)
        self_dma = pltpu.async_copy(table_hbm.at[sidx_v], srows_v, sem_s)

        fire(0, buf0, sem0)
        fire(1, buf1, sem1)

        def pair_body(jj, _):
            j = 2 * jj
            wait(buf0, sem0)

            @pl.when(jj < CHUNKS // 2 - 1)
            def _():
                fire(j + 2, buf0, sem0)

            accumulate(j, buf0)
            wait(buf1, sem1)

            @pl.when(jj < CHUNKS // 2 - 1)
            def _():
                fire(j + 3, buf1, sem1)

            accumulate(j + 1, buf1)
            return 0

        lax.fori_loop(0, CHUNKS // 2, pair_body, 0)

        pltpu.sync_copy(acc_v, out_neigh.at[pl.ds(gbase, GROUP)])
        self_dma.wait()
        pltpu.sync_copy(srows_v, out_self.at[pl.ds(gbase, GROUP)])
        return 0

    lax.fori_loop(0, N_GROUPS, group_body, 0)


@functools.partial(
    pl.kernel,
    mesh=plsc.VectorSubcoreMesh(core_axis_name="c", subcore_axis_name="s"),
    out_type=[
        jax.ShapeDtypeStruct((B, D), jnp.float32),
        jax.ShapeDtypeStruct((B, D), jnp.float32),
    ],
    scratch_types=[
        pltpu.VMEM((CHUNKS, CHUNK_N * DEG), jnp.int32),
        pltpu.VMEM((CHUNK_N * DEG, D), jnp.float32),
        pltpu.VMEM((CHUNK_N * DEG, D), jnp.float32),
        pltpu.VMEM((GROUP, D), jnp.float32),
        pltpu.VMEM((GROUP,), jnp.int32),
        pltpu.VMEM((GROUP, D), jnp.float32),
        pltpu.SemaphoreType.DMA,
        pltpu.SemaphoreType.DMA,
        pltpu.SemaphoreType.DMA,
    ],
)
def _sc_gather_mean(nb_hbm, nodes_hbm, table_hbm, out_self, out_neigh,
                    nidx_v, buf0, buf1, acc_v, sidx_v, srows_v,
                    sem0, sem1, sem_s):
    _sc_gather_mean_body(nb_hbm, nodes_hbm, table_hbm, out_self, out_neigh,
                         nidx_v, buf0, buf1, acc_v, sidx_v, srows_v,
                         sem0, sem1, sem_s)


def _tc_dense_body(s_ref, n_ref, w_ref, b_ref, g_ref, be_ref, out_ref):
    s = s_ref[...]
    n = n_ref[...]
    w = w_ref[...]
    lin = lax.dot_general(s, w[:, :D], (((1,), (1,)), ((), ())),
                          preferred_element_type=jnp.float32)
    lin = lin + lax.dot_general(n, w[:, D:], (((1,), (1,)), ((), ())),
                                preferred_element_type=jnp.float32)
    lin = lin + b_ref[...]
    mu = jnp.mean(lin, axis=0, keepdims=True)
    xc = lin - mu
    var = jnp.mean(xc * xc, axis=0, keepdims=True)
    inv = lax.rsqrt(var + 1e-5)
    out_ref[...] = jnp.maximum(xc * inv * g_ref[...] + be_ref[...], 0.0)


def kernel(nodes, neighbors, emb_table, W1, b1, gamma, beta):
    nb_chunked = neighbors.reshape(B // CHUNK_N, CHUNK_N * DEG)
    self_feats, neigh_mean = _sc_gather_mean(nb_chunked, nodes, emb_table)
    out = pl.pallas_call(
        _tc_dense_body,
        out_shape=jax.ShapeDtypeStruct((B, D), jnp.float32),
    )(self_feats, neigh_mean, W1,
      b1.reshape(1, D), gamma.reshape(1, D), beta.reshape(1, D))
    return out


# speedup vs baseline: 4.1583x; 1.0150x over previous
import functools

import jax
import jax.numpy as jnp
from jax import lax
from jax.experimental import pallas as pl
from jax.experimental.pallas import tpu as pltpu
from jax.experimental.pallas import tpu_sc as plsc

B = 16384
DEG = 32
D = 128

_info = plsc.get_sparse_core_info()
_NC, _NS = _info.num_cores, _info.num_subcores
NW = _NC * _NS
B_PER_W = B // NW
GROUP = 128
N_GROUPS = B_PER_W // GROUP
CHUNK_N = 4
CHUNKS = GROUP // CHUNK_N


def _sc_gather_mean_body(nb_hbm, nodes_hbm, table_hbm, out_self, out_neigh,
                         nidx_v, buf0, buf1, acc0, acc1, sidx_v,
                         srows0, srows1, sem0, sem1, sem_s, sem_i,
                         sem_fa, sem_fs):
    wid = lax.axis_index("s") * _NC + lax.axis_index("c")
    base = wid * B_PER_W
    accs = (acc0, acc1)
    srowss = (srows0, srows1)

    def fire(jrow, buf, sem):
        pltpu.async_copy(table_hbm.at[jrow], buf, sem)

    def wait(buf, sem):
        pltpu.make_async_copy(table_hbm.at[pl.ds(0, CHUNK_N * DEG)], buf,
                              sem).wait()

    def accumulate(j, buf, acc_v):
        def node_body(n, _):
            row0 = n * DEG
            for h in range(2):
                cols = [pl.ds((4 * h + c) * 16, 16) for c in range(4)]
                accs_ = [buf[row0, sl] for sl in cols]
                for r in range(1, DEG):
                    for c in range(4):
                        accs_[c] = accs_[c] + buf[row0 + r, cols[c]]
                for c in range(4):
                    acc_v[j * CHUNK_N + n, cols[c]] = accs_[c] * (1.0 / DEG)
            return 0

        lax.fori_loop(0, CHUNK_N, node_body, 0)

    def load_idx(gr):
        grow = pl.multiple_of((base + gr * GROUP) // CHUNK_N, CHUNKS)
        return pltpu.async_copy(nb_hbm.at[pl.ds(grow, CHUNKS)],
                                nidx_v.at[gr % 2], sem_i)

    def fire_self(gr):
        gbase = base + gr * GROUP
        pltpu.sync_copy(nodes_hbm.at[pl.ds(gbase, GROUP)], sidx_v)
        pltpu.async_copy(table_hbm.at[sidx_v], srowss[gr % 2], sem_s)

    load_idx(0).wait()
    fire_self(0)
    if N_GROUPS > 1:
        load_idx(1)

    for gr in range(N_GROUPS):
        acc_v = accs[gr % 2]
        srows_v = srowss[gr % 2]
        idx = nidx_v.at[gr % 2]

        if gr >= 2:
            pltpu.make_async_copy(table_hbm.at[pl.ds(0, GROUP)], acc_v,
                                  sem_fa).wait()

        fire(idx.at[0], buf0, sem0)
        fire(idx.at[1], buf1, sem1)

        def pair_body(jj, _):
            j = 2 * jj
            wait(buf0, sem0)

            @pl.when(jj < CHUNKS // 2 - 1)
            def _():
                fire(idx.at[j + 2], buf0, sem0)

            accumulate(j, buf0, acc_v)
            wait(buf1, sem1)

            @pl.when(jj < CHUNKS // 2 - 1)
            def _():
                fire(idx.at[j + 3], buf1, sem1)

            accumulate(j + 1, buf1, acc_v)
            return 0

        lax.fori_loop(0, CHUNKS // 2, pair_body, 0)

        gbase = base + gr * GROUP
        pltpu.async_copy(acc_v, out_neigh.at[pl.ds(gbase, GROUP)], sem_fa)
        pltpu.make_async_copy(table_hbm.at[pl.ds(0, GROUP)], srows_v,
                              sem_s).wait()
        pltpu.async_copy(srows_v, out_self.at[pl.ds(gbase, GROUP)], sem_fs)

        if gr + 2 < N_GROUPS:
            load_idx(gr + 2)
        if gr + 1 < N_GROUPS:
            pltpu.make_async_copy(nb_hbm.at[pl.ds(0, CHUNKS)],
                                  nidx_v.at[(gr + 1) % 2], sem_i).wait()
            if gr + 1 >= 2:
                pltpu.make_async_copy(table_hbm.at[pl.ds(0, GROUP)],
                                      srowss[(gr + 1) % 2], sem_fs).wait()
            fire_self(gr + 1)

    for _ in range(min(2, N_GROUPS)):
        pltpu.make_async_copy(table_hbm.at[pl.ds(0, GROUP)], acc0,
                              sem_fa).wait()
        pltpu.make_async_copy(table_hbm.at[pl.ds(0, GROUP)], srows0,
                              sem_fs).wait()


@functools.partial(
    pl.kernel,
    mesh=plsc.VectorSubcoreMesh(core_axis_name="c", subcore_axis_name="s"),
    out_type=[
        jax.ShapeDtypeStruct((B, D), jnp.float32),
        jax.ShapeDtypeStruct((B, D), jnp.float32),
    ],
    scratch_types=[
        pltpu.VMEM((2, CHUNKS, CHUNK_N * DEG), jnp.int32),
        pltpu.VMEM((CHUNK_N * DEG, D), jnp.float32),
        pltpu.VMEM((CHUNK_N * DEG, D), jnp.float32),
        pltpu.VMEM((GROUP, D), jnp.float32),
        pltpu.VMEM((GROUP, D), jnp.float32),
        pltpu.VMEM((GROUP,), jnp.int32),
        pltpu.VMEM((GROUP, D), jnp.float32),
        pltpu.VMEM((GROUP, D), jnp.float32),
        pltpu.SemaphoreType.DMA,
        pltpu.SemaphoreType.DMA,
        pltpu.SemaphoreType.DMA,
        pltpu.SemaphoreType.DMA,
        pltpu.SemaphoreType.DMA,
        pltpu.SemaphoreType.DMA,
    ],
)
def _sc_gather_mean(nb_hbm, nodes_hbm, table_hbm, out_self, out_neigh,
                    nidx_v, buf0, buf1, acc0, acc1, sidx_v,
                    srows0, srows1, sem0, sem1, sem_s, sem_i,
                    sem_fa, sem_fs):
    _sc_gather_mean_body(nb_hbm, nodes_hbm, table_hbm, out_self, out_neigh,
                         nidx_v, buf0, buf1, acc0, acc1, sidx_v,
                         srows0, srows1, sem0, sem1, sem_s, sem_i,
                         sem_fa, sem_fs)


def _tc_dense_body(s_ref, n_ref, w_ref, b_ref, g_ref, be_ref, out_ref):
    s = s_ref[...]
    n = n_ref[...]
    w = w_ref[...]
    lin = lax.dot_general(s, w[:, :D], (((1,), (1,)), ((), ())),
                          preferred_element_type=jnp.float32)
    lin = lin + lax.dot_general(n, w[:, D:], (((1,), (1,)), ((), ())),
                                preferred_element_type=jnp.float32)
    lin = lin + b_ref[...]
    mu = jnp.mean(lin, axis=0, keepdims=True)
    xc = lin - mu
    var = jnp.mean(xc * xc, axis=0, keepdims=True)
    inv = lax.rsqrt(var + 1e-5)
    out_ref[...] = jnp.maximum(xc * inv * g_ref[...] + be_ref[...], 0.0)


def kernel(nodes, neighbors, emb_table, W1, b1, gamma, beta):
    nb_chunked = neighbors.reshape(B // CHUNK_N, CHUNK_N * DEG)
    self_feats, neigh_mean = _sc_gather_mean(nb_chunked, nodes, emb_table)
    out = pl.pallas_call(
        _tc_dense_body,
        out_shape=jax.ShapeDtypeStruct((B, D), jnp.float32),
    )(self_feats, neigh_mean, W1,
      b1.reshape(1, D), gamma.reshape(1, D), beta.reshape(1, D))
    return out
